# baseline (speedup 1.0000x reference)
import functools
import math

import jax
import jax.numpy as jnp
from jax import lax
from jax.experimental import pallas as pl
from jax.experimental.pallas import tpu as pltpu
from jax.experimental.pallas import tpu_sc as plsc

EMB = 64
LANES = 16
NW = 32
BT = 128


def _make_sc_embed(bsz: int, seq: int, scale: float):
    mesh = plsc.VectorSubcoreMesh(core_axis_name="c", subcore_axis_name="s")
    nbt = bsz // BT
    bt_per_w = nbt // NW
    bw = bt_per_w * BT
    toks_per_w = bw * seq
    nchunk = seq * bt_per_w
    er = EMB // 8

    scratch = [
        pltpu.VMEM((toks_per_w,), jnp.int32),
        pltpu.VMEM((seq, bw), jnp.int32),
        pltpu.VMEM((BT, EMB), jnp.float32),
        pltpu.VMEM((er, 8, BT), jnp.float32),
        pltpu.SemaphoreType.DMA,
        pltpu.SemaphoreType.DMA,
    ]

    @functools.partial(
        pl.kernel,
        mesh=mesh,
        out_type=jax.ShapeDtypeStruct((seq, er, nbt, 8, BT), jnp.float32),
        scratch_types=scratch,
        compiler_params=pltpu.CompilerParams(
            use_tc_tiling_on_sc=False, needs_layout_passes=False),
    )
    def sc_embed(tokens_hbm, table_hbm, out_hbm, idx_v, idx_t, rows, obuf,
                 gsem, ssem):
        nc = lax.axis_size("c")
        wid = lax.axis_index("s") * nc + lax.axis_index("c")
        pltpu.sync_copy(tokens_hbm.at[pl.ds(wid * toks_per_w, toks_per_w)],
                        idx_v)

        iota16 = lax.iota(jnp.int32, 16)

        def tgroup(g, c):
            f = g * LANES + iota16
            vals = idx_v[pl.ds(g * LANES, LANES)]
            plsc.store_scatter(idx_t, [lax.rem(f, seq), lax.div(f, seq)],
                               vals)
            return c

        lax.fori_loop(0, toks_per_w // LANES, tgroup, 0, unroll=8)

        evecs = [LANES * v + iota16 for v in range(EMB // LANES)]
        etv = [lax.div(e, 8) for e in evecs]
        erv = [lax.rem(e, 8) for e in evecs]

        def chunk_body(t, c):
            s = lax.div(t, bt_per_w)
            btl = lax.rem(t, bt_per_w)
            pltpu.async_copy(
                table_hbm.at[idx_t.at[s, pl.ds(btl * BT, BT)]], rows,
                gsem).wait()

            def bc_loop(bc, c2):
                bcv = (iota16 & 0) + bc
                for v in range(EMB // LANES):
                    x = rows[bc, pl.ds(v * LANES, LANES)] * scale
                    plsc.store_scatter(obuf, [etv[v], erv[v], bcv], x)
                return c2

            lax.fori_loop(0, BT, bc_loop, 0, unroll=8)

            pltpu.sync_copy(
                obuf, out_hbm.at[s, pl.ds(0, er), wid * bt_per_w + btl])
            return c

        lax.fori_loop(0, nchunk, chunk_body, 0)

    return sc_embed


def kernel(tokens, table):
    bsz, seq = tokens.shape
    assert bsz % (NW * BT) == 0
    scale = math.sqrt(float(EMB))
    flat_tokens = tokens.reshape(-1).astype(jnp.int32)
    out5 = _make_sc_embed(bsz, seq, scale)(flat_tokens, table)
    return out5.transpose(2, 4, 0, 1, 3).reshape(bsz, seq, EMB)


# speedup vs baseline: 1.4657x; 1.4657x over previous
import functools
import math

import jax
import jax.numpy as jnp
from jax import lax
from jax.experimental import pallas as pl
from jax.experimental.pallas import tpu as pltpu
from jax.experimental.pallas import tpu_sc as plsc

EMB = 64
LANES = 16
NW = 32
BT = 128


def _make_sc_embed(bsz: int, seq: int, scale: float):
    mesh = plsc.VectorSubcoreMesh(core_axis_name="c", subcore_axis_name="s")
    nbt = bsz // BT
    bt_per_w = nbt // NW
    bw = bt_per_w * BT
    toks_per_w = bw * seq
    nchunk = seq * bt_per_w
    er = EMB // 8

    scratch = [
        pltpu.VMEM((toks_per_w,), jnp.int32),
        pltpu.VMEM((seq, bw + 8), jnp.int32),
        pltpu.VMEM((BT, EMB), jnp.float32),
        pltpu.VMEM((er, 8, BT + 1), jnp.float32),
        pltpu.SemaphoreType.DMA,
        pltpu.SemaphoreType.DMA,
    ]

    @functools.partial(
        pl.kernel,
        mesh=mesh,
        out_type=jax.ShapeDtypeStruct((seq, er, nbt, 8, BT), jnp.float32),
        scratch_types=scratch,
        compiler_params=pltpu.CompilerParams(
            use_tc_tiling_on_sc=False, needs_layout_passes=False),
    )
    def sc_embed(tokens_hbm, table_hbm, out_hbm, idx_v, idx_t, rows, obuf,
                 gsem, ssem):
        nc = lax.axis_size("c")
        wid = lax.axis_index("s") * nc + lax.axis_index("c")
        pltpu.sync_copy(tokens_hbm.at[pl.ds(wid * toks_per_w, toks_per_w)],
                        idx_v)

        iota16 = lax.iota(jnp.int32, 16)

        def tgroup(g, c):
            f = g * LANES + iota16
            vals = idx_v[pl.ds(g * LANES, LANES)]
            plsc.store_scatter(idx_t, [lax.rem(f, seq), lax.div(f, seq)],
                               vals)
            return c

        lax.fori_loop(0, toks_per_w // LANES, tgroup, 0, unroll=8)

        evecs = [LANES * v + iota16 for v in range(EMB // LANES)]
        etv = [lax.div(e, 8) for e in evecs]
        erv = [lax.rem(e, 8) for e in evecs]

        def chunk_body(t, c):
            s = lax.div(t, bt_per_w)
            btl = lax.rem(t, bt_per_w)
            pltpu.async_copy(
                table_hbm.at[idx_t.at[s, pl.ds(btl * BT, BT)]], rows,
                gsem).wait()

            def bc_loop(bc, c2):
                bcv = (iota16 & 0) + bc
                for v in range(EMB // LANES):
                    x = rows[bc, pl.ds(v * LANES, LANES)] * scale
                    plsc.store_scatter(obuf, [etv[v], erv[v], bcv], x)
                return c2

            lax.fori_loop(0, BT, bc_loop, 0, unroll=8)

            pltpu.sync_copy(
                obuf.at[pl.ds(0, er), pl.ds(0, 8), pl.ds(0, BT)],
                out_hbm.at[s, pl.ds(0, er), wid * bt_per_w + btl])
            return c

        lax.fori_loop(0, nchunk, chunk_body, 0)

    return sc_embed


def kernel(tokens, table):
    bsz, seq = tokens.shape
    assert bsz % (NW * BT) == 0
    scale = math.sqrt(float(EMB))
    flat_tokens = tokens.reshape(-1).astype(jnp.int32)
    out5 = _make_sc_embed(bsz, seq, scale)(flat_tokens, table)
    return out5.transpose(2, 4, 0, 1, 3).reshape(bsz, seq, EMB)


# speedup vs baseline: 1.7298x; 1.1802x over previous
import functools
import math

import jax
import jax.numpy as jnp
from jax import lax
from jax.experimental import pallas as pl
from jax.experimental.pallas import tpu as pltpu
from jax.experimental.pallas import tpu_sc as plsc

EMB = 64
LANES = 16
NW = 32
BT = 128


def _make_sc_embed(bsz: int, seq: int, scale: float):
    mesh = plsc.VectorSubcoreMesh(core_axis_name="c", subcore_axis_name="s")
    nbt = bsz // BT
    bt_per_w = nbt // NW
    bw = bt_per_w * BT
    toks_per_w = bw * seq
    nchunk = seq * bt_per_w
    er = EMB // 8

    scratch = [
        pltpu.VMEM((toks_per_w,), jnp.int32),
        pltpu.VMEM((seq, bw + 8), jnp.int32),
        pltpu.VMEM((BT, EMB), jnp.float32),
        pltpu.VMEM((BT, EMB), jnp.float32),
        pltpu.VMEM((er, 8, BT + 1), jnp.float32),
        pltpu.SemaphoreType.DMA,
        pltpu.SemaphoreType.DMA,
        pltpu.SemaphoreType.DMA,
    ]

    @functools.partial(
        pl.kernel,
        mesh=mesh,
        out_type=jax.ShapeDtypeStruct((seq, er, nbt, 8, BT), jnp.float32),
        scratch_types=scratch,
        compiler_params=pltpu.CompilerParams(
            use_tc_tiling_on_sc=False, needs_layout_passes=False),
    )
    def sc_embed(tokens_hbm, table_hbm, out_hbm, idx_v, idx_t, rows0, rows1,
                 obuf, gsem0, gsem1, ssem):
        rows = (rows0, rows1)
        gsem = (gsem0, gsem1)
        nc = lax.axis_size("c")
        wid = lax.axis_index("s") * nc + lax.axis_index("c")
        pltpu.sync_copy(tokens_hbm.at[pl.ds(wid * toks_per_w, toks_per_w)],
                        idx_v)

        iota16 = lax.iota(jnp.int32, 16)

        def tgroup(g, c):
            f = g * LANES + iota16
            vals = idx_v[pl.ds(g * LANES, LANES)]
            plsc.store_scatter(idx_t, [lax.rem(f, seq), lax.div(f, seq)],
                               vals)
            return c

        lax.fori_loop(0, toks_per_w // LANES, tgroup, 0, unroll=8)

        evecs = [LANES * v + iota16 for v in range(EMB // LANES)]
        etv = [lax.div(e, 8) for e in evecs]
        erv = [lax.rem(e, 8) for e in evecs]

        def fire_gather(t, b):
            s = lax.div(t, bt_per_w)
            btl = lax.rem(t, bt_per_w)
            pltpu.async_copy(
                table_hbm.at[idx_t.at[s, pl.ds(btl * BT, BT)]], rows[b],
                gsem[b])

        def wait_gather(b):
            pltpu.make_async_copy(
                table_hbm.at[idx_t.at[0, pl.ds(0, BT)]], rows[b],
                gsem[b]).wait()

        def do_chunk(t, b):
            s = lax.div(t, bt_per_w)
            btl = lax.rem(t, bt_per_w)
            wait_gather(b)

            def bc_loop(bc, c2):
                bcv = (iota16 & 0) + bc
                for v in range(EMB // LANES):
                    x = rows[b][bc, pl.ds(v * LANES, LANES)] * scale
                    plsc.store_scatter(obuf, [etv[v], erv[v], bcv], x)
                return c2

            lax.fori_loop(0, BT, bc_loop, 0, unroll=8)

            @pl.when(t + 2 < nchunk)
            def _refill():
                fire_gather(t + 2, b)

            pltpu.sync_copy(
                obuf.at[pl.ds(0, er), pl.ds(0, 8), pl.ds(0, BT)],
                out_hbm.at[s, pl.ds(0, er), wid * bt_per_w + btl])

        fire_gather(0, 0)
        fire_gather(1, 1)

        def pair_body(p, c):
            do_chunk(2 * p, 0)
            do_chunk(2 * p + 1, 1)
            return c

        lax.fori_loop(0, nchunk // 2, pair_body, 0)

    return sc_embed


def kernel(tokens, table):
    bsz, seq = tokens.shape
    assert bsz % (NW * BT) == 0
    scale = math.sqrt(float(EMB))
    flat_tokens = tokens.reshape(-1).astype(jnp.int32)
    out5 = _make_sc_embed(bsz, seq, scale)(flat_tokens, table)
    return out5.transpose(2, 4, 0, 1, 3).reshape(bsz, seq, EMB)


# speedup vs baseline: 1.8041x; 1.0429x over previous
import functools
import math

import jax
import jax.numpy as jnp
from jax import lax
from jax.experimental import pallas as pl
from jax.experimental.pallas import tpu as pltpu
from jax.experimental.pallas import tpu_sc as plsc

EMB = 64
LANES = 16
NW = 32
BT = 128


def _make_sc_embed(bsz: int, seq: int, scale: float):
    mesh = plsc.VectorSubcoreMesh(core_axis_name="c", subcore_axis_name="s")
    nbt = bsz // BT
    bt_per_w = nbt // NW
    bw = bt_per_w * BT
    toks_per_w = bw * seq
    nchunk = seq * bt_per_w
    er = EMB // 8

    scratch = [
        pltpu.VMEM((toks_per_w,), jnp.int32),
        pltpu.VMEM((seq, bw + 8), jnp.int32),
        pltpu.VMEM((BT, EMB), jnp.float32),
        pltpu.VMEM((BT, EMB), jnp.float32),
        pltpu.VMEM((er, 8, BT + 1), jnp.float32),
        pltpu.VMEM((er, 8, BT + 1), jnp.float32),
        pltpu.SemaphoreType.DMA,
        pltpu.SemaphoreType.DMA,
        pltpu.SemaphoreType.DMA,
        pltpu.SemaphoreType.DMA,
    ]

    @functools.partial(
        pl.kernel,
        mesh=mesh,
        out_type=jax.ShapeDtypeStruct((seq, er, nbt, 8, BT), jnp.float32),
        scratch_types=scratch,
        compiler_params=pltpu.CompilerParams(
            use_tc_tiling_on_sc=False, needs_layout_passes=False),
    )
    def sc_embed(tokens_hbm, table_hbm, out_hbm, idx_v, idx_t, rows0, rows1,
                 obuf0, obuf1, gsem0, gsem1, ssem0, ssem1):
        rows = (rows0, rows1)
        obufs = (obuf0, obuf1)
        gsem = (gsem0, gsem1)
        ssem = (ssem0, ssem1)
        nc = lax.axis_size("c")
        wid = lax.axis_index("s") * nc + lax.axis_index("c")
        pltpu.sync_copy(tokens_hbm.at[pl.ds(wid * toks_per_w, toks_per_w)],
                        idx_v)

        iota16 = lax.iota(jnp.int32, 16)

        def tgroup(g, c):
            f = g * LANES + iota16
            vals = idx_v[pl.ds(g * LANES, LANES)]
            plsc.store_scatter(idx_t, [lax.rem(f, seq), lax.div(f, seq)],
                               vals)
            return c

        lax.fori_loop(0, toks_per_w // LANES, tgroup, 0, unroll=8)

        evecs = [LANES * v + iota16 for v in range(EMB // LANES)]
        etv = [lax.div(e, 8) for e in evecs]
        erv = [lax.rem(e, 8) for e in evecs]

        def fire_gather(t, b):
            s = lax.div(t, bt_per_w)
            btl = lax.rem(t, bt_per_w)
            pltpu.async_copy(
                table_hbm.at[idx_t.at[s, pl.ds(btl * BT, BT)]], rows[b],
                gsem[b])

        def wait_gather(b):
            pltpu.make_async_copy(
                table_hbm.at[idx_t.at[0, pl.ds(0, BT)]], rows[b],
                gsem[b]).wait()

        def do_chunk(t, b):
            s = lax.div(t, bt_per_w)
            btl = lax.rem(t, bt_per_w)
            wait_gather(b)

            @pl.when(t >= 2)
            def _drain_store():
                pltpu.make_async_copy(
                    obufs[b].at[pl.ds(0, er), pl.ds(0, 8), pl.ds(0, BT)],
                    out_hbm.at[0, pl.ds(0, er), 0], ssem[b]).wait()

            def bc_loop(bc, c2):
                bcv = (iota16 & 0) + bc
                for v in range(EMB // LANES):
                    x = rows[b][bc, pl.ds(v * LANES, LANES)] * scale
                    plsc.store_scatter(obufs[b], [etv[v], erv[v], bcv], x)
                return c2

            lax.fori_loop(0, BT, bc_loop, 0, unroll=16)

            @pl.when(t + 2 < nchunk)
            def _refill():
                fire_gather(t + 2, b)

            pltpu.async_copy(
                obufs[b].at[pl.ds(0, er), pl.ds(0, 8), pl.ds(0, BT)],
                out_hbm.at[s, pl.ds(0, er), wid * bt_per_w + btl], ssem[b])

        fire_gather(0, 0)
        fire_gather(1, 1)

        def pair_body(p, c):
            do_chunk(2 * p, 0)
            do_chunk(2 * p + 1, 1)
            return c

        lax.fori_loop(0, nchunk // 2, pair_body, 0)

        for b in range(2):
            pltpu.make_async_copy(
                obufs[b].at[pl.ds(0, er), pl.ds(0, 8), pl.ds(0, BT)],
                out_hbm.at[0, pl.ds(0, er), 0], ssem[b]).wait()

    return sc_embed


def kernel(tokens, table):
    bsz, seq = tokens.shape
    assert bsz % (NW * BT) == 0
    scale = math.sqrt(float(EMB))
    flat_tokens = tokens.reshape(-1).astype(jnp.int32)
    out5 = _make_sc_embed(bsz, seq, scale)(flat_tokens, table)
    return out5.transpose(2, 4, 0, 1, 3).reshape(bsz, seq, EMB)
